# baseline (speedup 1.0000x reference)
import jax
import jax.numpy as jnp
import numpy as np
from jax import lax
from jax.experimental import pallas as pl
from jax.experimental.pallas import tpu as pltpu
from jax.experimental.pallas import tpu_sc as plsc

INPUT_DIM = 512
NUM_ELMT = 64
RATIO = 1.2
BOUND = 50.0
N = 65536

L = 16
NC = 2
NS = 16
NW = NC * NS
COLS = INPUT_DIM // NW
R = 2048
NCHUNK = N // R
YR_ROWS = 128


def _mesh_tables():
    one_step = BOUND * (RATIO - 1.0) / (RATIO ** (NUM_ELMT / 2) - 1.0)
    idx = np.arange(-NUM_ELMT // 2, NUM_ELMT // 2 + 1).astype(np.float64)
    sign = np.sign(idx)
    mesh = (RATIO ** np.abs(idx) - 1.0) / (RATIO - 1.0) * one_step * sign
    mesh_norm = (mesh + BOUND) / 2.0 / BOUND
    mesh_norm = np.concatenate([[0.0], mesh_norm[1:-1], [1.0]])
    elmt_size = mesh_norm[1:] - mesh_norm[:-1]
    return mesh_norm.astype(np.float32), elmt_size.astype(np.float32)


_MESH_NORM, _ELMT_SIZE = _mesh_tables()
_CF = (_ELMT_SIZE[:-1] + _ELMT_SIZE[1:]) / np.float32(2.0)
_C1 = np.float32(1.0) - _ELMT_SIZE[0]
_XS64 = (np.float32(100.0) * _MESH_NORM[:NUM_ELMT] - np.float32(50.0))
_H64 = np.float32(2.0) * _ELMT_SIZE


def _body(x_hbm, logp_hbm, xs_hbm, h_hbm, out_hbm,
          logp_v, yr_tab, pdf_tab, a_tab, b_tab, xs_tab, h_tab, xbuf, obuf):
    wid = lax.axis_index("s") * NC + lax.axis_index("c")
    c0 = wid * COLS
    lanes = lax.iota(jnp.int32, L)
    lane_lo = lanes + L
    lane_hi = lanes + NUM_ELMT * L

    pltpu.sync_copy(logp_hbm.at[:, pl.ds(c0, COLS)], logp_v)
    pltpu.sync_copy(xs_hbm, xs_tab)
    pltpu.sync_copy(h_hbm, h_tab)

    S = jnp.zeros((L,), jnp.float32)
    for m in range(NUM_ELMT - 1):
        wv = jnp.exp(logp_v[m, :])
        pdf_tab[pl.ds((m + 1) * L, L)] = wv
        S = S + wv * float(_CF[m])
    inv = float(_C1) / S
    one_v = jnp.ones((L,), jnp.float32)
    pdf_tab[pl.ds(0, L)] = one_v
    pdf_tab[pl.ds(NUM_ELMT * L, L)] = one_v
    for m in range(NUM_ELMT - 1):
        pdf_tab[pl.ds((m + 1) * L, L)] = pdf_tab[pl.ds((m + 1) * L, L)] * inv
    yr_tab[pl.ds(0, L)] = jnp.full((L,), -50.0, jnp.float32)
    F = jnp.zeros((L,), jnp.float32)
    prev = pdf_tab[pl.ds(0, L)]
    for s in range(NUM_ELMT):
        cur = pdf_tab[pl.ds((s + 1) * L, L)]
        a_tab[pl.ds(s * L, L)] = prev * float(_ELMT_SIZE[s])
        b_tab[pl.ds(s * L, L)] = (cur - prev) * float(0.02 * _ELMT_SIZE[s])
        if s < NUM_ELMT - 1:
            F = F + (prev + cur) * float(0.5 * _ELMT_SIZE[s])
            yr_tab[pl.ds((s + 1) * L, L)] = F * 100.0 - 50.0
        prev = cur
    yr_tab[pl.ds(NUM_ELMT * L, L)] = jnp.full((L,), 50.0, jnp.float32)
    inf_v = jnp.full((L,), jnp.inf, jnp.float32)
    for m in range(NUM_ELMT + 1, YR_ROWS):
        yr_tab[pl.ds(m * L, L)] = inf_v

    def chunk_body(t, carry):
        r0 = t * R
        pltpu.sync_copy(x_hbm.at[pl.ds(r0, R), pl.ds(c0, COLS)], xbuf)

        @plsc.parallel_loop(0, R, step=1, unroll=6)
        def row_body(i):
            xv = xbuf[i, :]
            K = lanes
            for b in (64, 32, 16, 8, 4, 2, 1):
                val = plsc.load_gather(yr_tab, [K + ((b - 1) * L)])
                K = jnp.where(val < xv, K + b * L, K)
            cover = (K >= L) & (K < (NUM_ELMT + 1) * L)
            Km1 = jnp.minimum(jnp.maximum(K, lane_lo), lane_hi) - L
            ykx = plsc.load_gather(yr_tab, [Km1])
            A = plsc.load_gather(a_tab, [Km1])
            B = plsc.load_gather(b_tab, [Km1])
            H = plsc.load_gather(h_tab, [Km1])
            XS = plsc.load_gather(xs_tab, [Km1])
            xr = xv - ykx
            dm = jnp.maximum(B * xr + A * A, 1e-30)
            iv = 0x5F3759DF - lax.shift_right_logical(
                lax.bitcast_convert_type(dm, jnp.int32), 1)
            rs = lax.bitcast_convert_type(iv, jnp.float32)
            hd = 0.5 * dm
            rs = rs * (1.5 - hd * rs * rs)
            rs = rs * (1.5 - hd * rs * rs)
            tp = (H * xr) / (dm * rs + A) + XS
            obuf[i, :] = jnp.where(cover, tp, xv)

        pltpu.sync_copy(obuf, out_hbm.at[pl.ds(r0, R), pl.ds(c0, COLS)])
        return carry

    lax.fori_loop(0, NCHUNK, chunk_body, 0)


_sc_call = pl.kernel(
    _body,
    out_type=jax.ShapeDtypeStruct((N, INPUT_DIM), jnp.float32),
    mesh=plsc.VectorSubcoreMesh(core_axis_name="c", subcore_axis_name="s"),
    compiler_params=pltpu.CompilerParams(
        use_tc_tiling_on_sc=False, needs_layout_passes=False),
    scratch_types=[
        pltpu.VMEM((NUM_ELMT - 1, COLS), jnp.float32),
        pltpu.VMEM((YR_ROWS * L,), jnp.float32),
        pltpu.VMEM(((NUM_ELMT + 1) * L,), jnp.float32),
        pltpu.VMEM((NUM_ELMT * L,), jnp.float32),
        pltpu.VMEM((NUM_ELMT * L,), jnp.float32),
        pltpu.VMEM((NUM_ELMT * L,), jnp.float32),
        pltpu.VMEM((NUM_ELMT * L,), jnp.float32),
        pltpu.VMEM((R, COLS), jnp.float32),
        pltpu.VMEM((R, COLS), jnp.float32),
    ],
)


def kernel(x, log_p):
    xs_c = jnp.asarray(np.tile(_XS64[:, None], (1, L)).reshape(-1))
    h_c = jnp.asarray(np.tile(_H64[:, None], (1, L)).reshape(-1))
    return _sc_call(x, log_p, xs_c, h_c)


# speedup vs baseline: 1.1718x; 1.1718x over previous
import jax
import jax.numpy as jnp
import numpy as np
from jax import lax
from jax.experimental import pallas as pl
from jax.experimental.pallas import tpu as pltpu
from jax.experimental.pallas import tpu_sc as plsc

INPUT_DIM = 512
NUM_ELMT = 64
RATIO = 1.2
BOUND = 50.0
N = 65536

L = 16
NC = 2
NS = 16
NW = NC * NS
COLS = INPUT_DIM // NW
R = 2048
NCHUNK = N // R
YR_ROWS = 128


def _mesh_tables():
    one_step = BOUND * (RATIO - 1.0) / (RATIO ** (NUM_ELMT / 2) - 1.0)
    idx = np.arange(-NUM_ELMT // 2, NUM_ELMT // 2 + 1).astype(np.float64)
    sign = np.sign(idx)
    mesh = (RATIO ** np.abs(idx) - 1.0) / (RATIO - 1.0) * one_step * sign
    mesh_norm = (mesh + BOUND) / 2.0 / BOUND
    mesh_norm = np.concatenate([[0.0], mesh_norm[1:-1], [1.0]])
    elmt_size = mesh_norm[1:] - mesh_norm[:-1]
    return mesh_norm.astype(np.float32), elmt_size.astype(np.float32)


_MESH_NORM, _ELMT_SIZE = _mesh_tables()
_CF = (_ELMT_SIZE[:-1] + _ELMT_SIZE[1:]) / np.float32(2.0)
_C1 = np.float32(1.0) - _ELMT_SIZE[0]
_XS64 = (np.float32(100.0) * _MESH_NORM[:NUM_ELMT] - np.float32(50.0))
_H64 = np.float32(2.0) * _ELMT_SIZE


def _body(x_hbm, logp_hbm, xs_hbm, h_hbm, out_hbm,
          logp_v, yr_tab, pdf_tab, a_tab, b_tab, xs_tab, h_tab, xbuf, obuf):
    wid = lax.axis_index("s") * NC + lax.axis_index("c")
    c0 = wid * COLS
    lanes = lax.iota(jnp.int32, L)
    lane_lo = lanes + L
    lane_hi = lanes + NUM_ELMT * L

    pltpu.sync_copy(logp_hbm.at[:, pl.ds(c0, COLS)], logp_v)
    pltpu.sync_copy(xs_hbm, xs_tab)
    pltpu.sync_copy(h_hbm, h_tab)

    S = jnp.zeros((L,), jnp.float32)
    for m in range(NUM_ELMT - 1):
        wv = jnp.exp(logp_v[m, :])
        pdf_tab[pl.ds((m + 1) * L, L)] = wv
        S = S + wv * float(_CF[m])
    inv = float(_C1) / S
    one_v = jnp.ones((L,), jnp.float32)
    pdf_tab[pl.ds(0, L)] = one_v
    pdf_tab[pl.ds(NUM_ELMT * L, L)] = one_v
    for m in range(NUM_ELMT - 1):
        pdf_tab[pl.ds((m + 1) * L, L)] = pdf_tab[pl.ds((m + 1) * L, L)] * inv
    yr_tab[pl.ds(0, L)] = jnp.full((L,), -50.0, jnp.float32)
    F = jnp.zeros((L,), jnp.float32)
    prev = pdf_tab[pl.ds(0, L)]
    for s in range(NUM_ELMT):
        cur = pdf_tab[pl.ds((s + 1) * L, L)]
        a_tab[pl.ds(s * L, L)] = prev * float(_ELMT_SIZE[s])
        b_tab[pl.ds(s * L, L)] = (cur - prev) * float(0.02 * _ELMT_SIZE[s])
        if s < NUM_ELMT - 1:
            F = F + (prev + cur) * float(0.5 * _ELMT_SIZE[s])
            yr_tab[pl.ds((s + 1) * L, L)] = F * 100.0 - 50.0
        prev = cur
    yr_tab[pl.ds(NUM_ELMT * L, L)] = jnp.full((L,), 50.0, jnp.float32)
    inf_v = jnp.full((L,), jnp.inf, jnp.float32)
    for m in range(NUM_ELMT + 1, YR_ROWS):
        yr_tab[pl.ds(m * L, L)] = inf_v

    def chunk_body(t, carry):
        r0 = t * R
        pltpu.sync_copy(x_hbm.at[pl.ds(r0, R), pl.ds(c0, COLS)], xbuf)

        @plsc.parallel_loop(0, R, step=1, unroll=4)
        def row_body(i):
            xv = xbuf[i, :]
            K = lanes
            for b in (64, 32, 16, 8, 4, 2, 1):
                val = plsc.load_gather(yr_tab, [K + ((b - 1) * L)])
                K = jnp.where(val < xv, K + b * L, K)
            cover = (K >= L) & (K < (NUM_ELMT + 1) * L)
            Km1 = jnp.minimum(jnp.maximum(K, lane_lo), lane_hi) - L
            ykx = plsc.load_gather(yr_tab, [Km1])
            A = plsc.load_gather(a_tab, [Km1])
            B = plsc.load_gather(b_tab, [Km1])
            H = plsc.load_gather(h_tab, [Km1])
            XS = plsc.load_gather(xs_tab, [Km1])
            xr = xv - ykx
            dm = jnp.maximum(B * xr + A * A, 1e-30)
            iv = 0x5F3759DF - lax.shift_right_logical(
                lax.bitcast_convert_type(dm, jnp.int32), 1)
            rs = lax.bitcast_convert_type(iv, jnp.float32)
            hd = 0.5 * dm
            rs = rs * (1.5 - hd * rs * rs)
            rs = rs * (1.5 - hd * rs * rs)
            tp = (H * xr) / (dm * rs + A) + XS
            obuf[i, :] = jnp.where(cover, tp, xv)

        pltpu.sync_copy(obuf, out_hbm.at[pl.ds(r0, R), pl.ds(c0, COLS)])
        return carry

    lax.fori_loop(0, NCHUNK, chunk_body, 0)


_sc_call = pl.kernel(
    _body,
    out_type=jax.ShapeDtypeStruct((N, INPUT_DIM), jnp.float32),
    mesh=plsc.VectorSubcoreMesh(core_axis_name="c", subcore_axis_name="s"),
    compiler_params=pltpu.CompilerParams(
        use_tc_tiling_on_sc=False, needs_layout_passes=False),
    scratch_types=[
        pltpu.VMEM((NUM_ELMT - 1, COLS), jnp.float32),
        pltpu.VMEM((YR_ROWS * L,), jnp.float32),
        pltpu.VMEM(((NUM_ELMT + 1) * L,), jnp.float32),
        pltpu.VMEM((NUM_ELMT * L,), jnp.float32),
        pltpu.VMEM((NUM_ELMT * L,), jnp.float32),
        pltpu.VMEM((NUM_ELMT * L,), jnp.float32),
        pltpu.VMEM((NUM_ELMT * L,), jnp.float32),
        pltpu.VMEM((R, COLS), jnp.float32),
        pltpu.VMEM((R, COLS), jnp.float32),
    ],
)


def kernel(x, log_p):
    xs_c = jnp.asarray(np.tile(_XS64[:, None], (1, L)).reshape(-1))
    h_c = jnp.asarray(np.tile(_H64[:, None], (1, L)).reshape(-1))
    return _sc_call(x, log_p, xs_c, h_c)


# speedup vs baseline: 1.5824x; 1.3504x over previous
import jax
import jax.numpy as jnp
import numpy as np
from jax import lax
from jax.experimental import pallas as pl
from jax.experimental.pallas import tpu as pltpu
from jax.experimental.pallas import tpu_sc as plsc

INPUT_DIM = 512
NUM_ELMT = 64
RATIO = 1.2
BOUND = 50.0
N = 65536

L = 16
NC = 2
NS = 16
NW = NC * NS
COLS = INPUT_DIM // NW
R = 1024
NCHUNK = N // R
YR_ROWS = 128
KROWS = NUM_ELMT + 2


def _mesh_tables():
    one_step = BOUND * (RATIO - 1.0) / (RATIO ** (NUM_ELMT / 2) - 1.0)
    idx = np.arange(-NUM_ELMT // 2, NUM_ELMT // 2 + 1).astype(np.float64)
    sign = np.sign(idx)
    mesh = (RATIO ** np.abs(idx) - 1.0) / (RATIO - 1.0) * one_step * sign
    mesh_norm = (mesh + BOUND) / 2.0 / BOUND
    mesh_norm = np.concatenate([[0.0], mesh_norm[1:-1], [1.0]])
    elmt_size = mesh_norm[1:] - mesh_norm[:-1]
    return mesh_norm.astype(np.float32), elmt_size.astype(np.float32)


_MESH_NORM, _ELMT_SIZE = _mesh_tables()
_CF = (_ELMT_SIZE[:-1] + _ELMT_SIZE[1:]) / np.float32(2.0)
_C1 = np.float32(1.0) - _ELMT_SIZE[0]

_XS66 = np.concatenate([[np.float32(0.0)],
                        np.float32(100.0) * _MESH_NORM[:NUM_ELMT] - np.float32(50.0),
                        [np.float32(0.0)]]).astype(np.float32)
_H66 = np.concatenate([[np.float32(1.0)], np.float32(2.0) * _ELMT_SIZE,
                       [np.float32(1.0)]]).astype(np.float32)


def _body(x_hbm, logp_hbm, xs_hbm, h_hbm, out_hbm,
          logp_v, yr_tab, pdf_tab, yk_tab, a_tab, b_tab, xs_tab, h_tab,
          xbufA, xbufB, obufA, obufB, isemA, isemB, osemA, osemB):
    wid = lax.axis_index("s") * NC + lax.axis_index("c")
    c0 = wid * COLS
    lanes = lax.iota(jnp.int32, L)

    pltpu.sync_copy(logp_hbm.at[:, pl.ds(c0, COLS)], logp_v)
    pltpu.sync_copy(xs_hbm, xs_tab)
    pltpu.sync_copy(h_hbm, h_tab)

    S = jnp.zeros((L,), jnp.float32)
    for m in range(NUM_ELMT - 1):
        wv = jnp.exp(logp_v[m, :])
        pdf_tab[pl.ds((m + 1) * L, L)] = wv
        S = S + wv * float(_CF[m])
    inv = float(_C1) / S
    one_v = jnp.ones((L,), jnp.float32)
    zero_v = jnp.zeros((L,), jnp.float32)
    half_v = jnp.full((L,), 0.5, jnp.float32)
    pdf_tab[pl.ds(0, L)] = one_v
    pdf_tab[pl.ds(NUM_ELMT * L, L)] = one_v
    for m in range(NUM_ELMT - 1):
        pdf_tab[pl.ds((m + 1) * L, L)] = pdf_tab[pl.ds((m + 1) * L, L)] * inv
    yr_tab[pl.ds(0, L)] = jnp.full((L,), -50.0, jnp.float32)
    yk_tab[pl.ds(0, L)] = zero_v
    yk_tab[pl.ds((KROWS - 1) * L, L)] = zero_v
    b_tab[pl.ds(0, L)] = zero_v
    b_tab[pl.ds((KROWS - 1) * L, L)] = zero_v
    a_tab[pl.ds(0, L)] = half_v
    a_tab[pl.ds((KROWS - 1) * L, L)] = half_v
    yk_tab[pl.ds(L, L)] = jnp.full((L,), -50.0, jnp.float32)
    F = jnp.zeros((L,), jnp.float32)
    prev = pdf_tab[pl.ds(0, L)]
    for s in range(NUM_ELMT):
        cur = pdf_tab[pl.ds((s + 1) * L, L)]
        a_tab[pl.ds((s + 1) * L, L)] = prev * float(_ELMT_SIZE[s])
        b_tab[pl.ds((s + 1) * L, L)] = (cur - prev) * float(0.02 * _ELMT_SIZE[s])
        if s < NUM_ELMT - 1:
            F = F + (prev + cur) * float(0.5 * _ELMT_SIZE[s])
            yrow = F * 100.0 - 50.0
            yr_tab[pl.ds((s + 1) * L, L)] = yrow
            yk_tab[pl.ds((s + 2) * L, L)] = yrow
        prev = cur
    yr_tab[pl.ds(NUM_ELMT * L, L)] = jnp.full((L,), 50.0, jnp.float32)
    inf_v = jnp.full((L,), jnp.inf, jnp.float32)
    for m in range(NUM_ELMT + 1, YR_ROWS):
        yr_tab[pl.ds(m * L, L)] = inf_v

    def start_in(t, buf, sem):
        pltpu.async_copy(x_hbm.at[pl.ds(t * R, R), pl.ds(c0, COLS)], buf, sem)

    def wait_in(t, buf, sem):
        pltpu.make_async_copy(x_hbm.at[pl.ds(t * R, R), pl.ds(c0, COLS)],
                              buf, sem).wait()

    def start_out(t, buf, sem):
        pltpu.async_copy(buf, out_hbm.at[pl.ds(t * R, R), pl.ds(c0, COLS)], sem)

    def wait_out(t, buf, sem):
        pltpu.make_async_copy(buf, out_hbm.at[pl.ds(t * R, R), pl.ds(c0, COLS)],
                              sem).wait()

    def compute(xbuf, obuf):
        @plsc.parallel_loop(0, R, step=1, unroll=4)
        def row_body(i):
            xv = xbuf[i, :]
            K = lanes
            for b in (64, 32, 16, 8, 4, 2, 1):
                val = plsc.load_gather(yr_tab, [K + ((b - 1) * L)])
                K = jnp.where(val < xv, K + b * L, K)
            ykx = plsc.load_gather(yk_tab, [K])
            A = plsc.load_gather(a_tab, [K])
            B = plsc.load_gather(b_tab, [K])
            H = plsc.load_gather(h_tab, [K])
            XS = plsc.load_gather(xs_tab, [K])
            xr = xv - ykx
            dm = jnp.maximum(B * xr + A * A, 1e-30)
            iv = 0x1F3D1DF5 + lax.shift_right_logical(
                lax.bitcast_convert_type(dm, jnp.int32), 1)
            s0 = lax.bitcast_convert_type(iv, jnp.float32)
            s1 = s0 + (0.25 * dm) / s0
            obuf[i, :] = (H * xr) / (s1 + A) + XS

    xb = (xbufA, xbufB)
    ob = (obufA, obufB)
    isem = (isemA, isemB)
    osem = (osemA, osemB)
    start_in(0, xb[0], isem[0])

    def chunk_pair(g, carry):
        t0 = g * 2
        for ph in (0, 1):
            t = t0 + ph
            nxt = t + 1

            @pl.when(nxt < NCHUNK)
            def _():
                start_in(nxt, xb[1 - ph], isem[1 - ph])

            wait_in(t, xb[ph], isem[ph])

            @pl.when(t >= 2)
            def _():
                wait_out(t, ob[ph], osem[ph])

            compute(xb[ph], ob[ph])
            start_out(t, ob[ph], osem[ph])
        return carry

    lax.fori_loop(0, NCHUNK // 2, chunk_pair, 0)
    wait_out(NCHUNK - 2, ob[0], osem[0])
    wait_out(NCHUNK - 1, ob[1], osem[1])


_sc_call = pl.kernel(
    _body,
    out_type=jax.ShapeDtypeStruct((N, INPUT_DIM), jnp.float32),
    mesh=plsc.VectorSubcoreMesh(core_axis_name="c", subcore_axis_name="s"),
    compiler_params=pltpu.CompilerParams(
        use_tc_tiling_on_sc=False, needs_layout_passes=False),
    scratch_types=[
        pltpu.VMEM((NUM_ELMT - 1, COLS), jnp.float32),
        pltpu.VMEM((YR_ROWS * L,), jnp.float32),
        pltpu.VMEM(((NUM_ELMT + 1) * L,), jnp.float32),
        pltpu.VMEM((KROWS * L,), jnp.float32),
        pltpu.VMEM((KROWS * L,), jnp.float32),
        pltpu.VMEM((KROWS * L,), jnp.float32),
        pltpu.VMEM((KROWS * L,), jnp.float32),
        pltpu.VMEM((KROWS * L,), jnp.float32),
        pltpu.VMEM((R, COLS), jnp.float32),
        pltpu.VMEM((R, COLS), jnp.float32),
        pltpu.VMEM((R, COLS), jnp.float32),
        pltpu.VMEM((R, COLS), jnp.float32),
        pltpu.SemaphoreType.DMA,
        pltpu.SemaphoreType.DMA,
        pltpu.SemaphoreType.DMA,
        pltpu.SemaphoreType.DMA,
    ],
)


def kernel(x, log_p):
    xs_c = jnp.asarray(np.tile(_XS66[:, None], (1, L)).reshape(-1))
    h_c = jnp.asarray(np.tile(_H66[:, None], (1, L)).reshape(-1))
    return _sc_call(x, log_p, xs_c, h_c)


# speedup vs baseline: 2.1217x; 1.3408x over previous
import jax
import jax.numpy as jnp
import numpy as np
from jax import lax
from jax.experimental import pallas as pl
from jax.experimental.pallas import tpu as pltpu
from jax.experimental.pallas import tpu_sc as plsc

INPUT_DIM = 512
NUM_ELMT = 64
RATIO = 1.2
BOUND = 50.0
N = 65536

L = 16
NC = 2
NS = 16
NW = NC * NS
NCB = 4
NRG = NW // NCB
CB = INPUT_DIM // NCB
RW = N // NRG
Rc = 128
NCHUNK = RW // Rc
VPC = Rc * (CB // L)
YR_ROWS = 96
KROWS = NUM_ELMT + 2


def _mesh_tables():
    one_step = BOUND * (RATIO - 1.0) / (RATIO ** (NUM_ELMT / 2) - 1.0)
    idx = np.arange(-NUM_ELMT // 2, NUM_ELMT // 2 + 1).astype(np.float64)
    sign = np.sign(idx)
    mesh = (RATIO ** np.abs(idx) - 1.0) / (RATIO - 1.0) * one_step * sign
    mesh_norm = (mesh + BOUND) / 2.0 / BOUND
    mesh_norm = np.concatenate([[0.0], mesh_norm[1:-1], [1.0]])
    elmt_size = mesh_norm[1:] - mesh_norm[:-1]
    return mesh_norm.astype(np.float32), elmt_size.astype(np.float32)


_MESH_NORM, _ELMT_SIZE = _mesh_tables()
_CF = (_ELMT_SIZE[:-1] + _ELMT_SIZE[1:]) / np.float32(2.0)
_C1 = np.float32(1.0) - _ELMT_SIZE[0]

_XS66 = np.concatenate([[np.float32(0.0)],
                        np.float32(100.0) * _MESH_NORM[:NUM_ELMT] - np.float32(50.0),
                        [np.float32(0.0)]]).astype(np.float32)
_H66 = np.concatenate([[np.float32(1.0)], np.float32(2.0) * _ELMT_SIZE,
                       [np.float32(1.0)]]).astype(np.float32)


def _body(x_hbm, logp_hbm, xs_hbm, h_hbm, out_hbm,
          yr_tab, yk_tab, a_tab, b_tab, xs_tab, h_tab,
          xbufA, xbufB, obufA, obufB, isemA, isemB, osemA, osemB):
    wid = lax.axis_index("s") * NC + lax.axis_index("c")
    cb = wid & 3
    rg = lax.shift_right_logical(wid, 2)
    c0 = pl.multiple_of(cb * CB, CB)
    rbase = pl.multiple_of(rg * RW, 8)
    lanes = lax.iota(jnp.int32, L)

    pltpu.sync_copy(logp_hbm.at[:, pl.ds(c0, CB)], obufA.at[pl.ds(0, 63), :])
    pltpu.sync_copy(xs_hbm, xs_tab)
    pltpu.sync_copy(h_hbm, h_tab)

    one_v = jnp.ones((L,), jnp.float32)
    zero_v = jnp.zeros((L,), jnp.float32)

    def prep(cv, carry):
        base = pl.multiple_of(cv * L, L)
        S = jnp.zeros((L,), jnp.float32)
        for m in range(NUM_ELMT - 1):
            wv = jnp.exp(obufA[m, pl.ds(base, L)])
            b_tab[pl.ds((m + 1) * CB + base, L)] = wv
            S = S + wv * float(_CF[m])
        inv = float(_C1) / S
        yk_tab[pl.ds(0 * CB + base, L)] = zero_v
        yk_tab[pl.ds((KROWS - 1) * CB + base, L)] = zero_v
        b_tab[pl.ds(0 * CB + base, L)] = zero_v
        a_tab[pl.ds(0 * CB + base, L)] = jnp.full((L,), 0.5, jnp.float32)
        a_tab[pl.ds((KROWS - 1) * CB + base, L)] = jnp.full((L,), 0.5, jnp.float32)
        m50 = jnp.full((L,), -50.0, jnp.float32)
        yr_tab[pl.ds(0 * CB + base, L)] = m50
        yk_tab[pl.ds(1 * CB + base, L)] = m50
        F = jnp.zeros((L,), jnp.float32)
        prev = one_v
        for s in range(NUM_ELMT):
            if s < NUM_ELMT - 1:
                cur = b_tab[pl.ds((s + 1) * CB + base, L)] * inv
            else:
                cur = one_v
            a_tab[pl.ds((s + 1) * CB + base, L)] = prev * float(_ELMT_SIZE[s])
            b_tab[pl.ds((s + 1) * CB + base, L)] = (
                (cur - prev) * float(0.02 * _ELMT_SIZE[s]))
            if s < NUM_ELMT - 1:
                F = F + (prev + cur) * float(0.5 * _ELMT_SIZE[s])
                yrow = F * 100.0 - 50.0
                yr_tab[pl.ds((s + 1) * CB + base, L)] = yrow
                yk_tab[pl.ds((s + 2) * CB + base, L)] = yrow
            prev = cur
        yr_tab[pl.ds(NUM_ELMT * CB + base, L)] = jnp.full((L,), 50.0, jnp.float32)
        b_tab[pl.ds((KROWS - 1) * CB + base, L)] = zero_v
        return carry

    lax.fori_loop(0, CB // L, prep, 0)

    inf_v = jnp.full((L,), jnp.inf, jnp.float32)

    def pad(i, carry):
        yr_tab[pl.ds(pl.multiple_of(i * L, L), L)] = inf_v
        return carry

    lax.fori_loop((NUM_ELMT + 1) * (CB // L), YR_ROWS * (CB // L), pad, 0)

    def start_in(t, buf, sem):
        pltpu.async_copy(
            x_hbm.at[pl.ds(rbase + t * Rc, Rc), pl.ds(c0, CB)], buf, sem)

    def wait_in(t, buf, sem):
        pltpu.make_async_copy(
            x_hbm.at[pl.ds(rbase + t * Rc, Rc), pl.ds(c0, CB)], buf, sem).wait()

    def start_out(t, buf, sem):
        pltpu.async_copy(
            buf, out_hbm.at[pl.ds(rbase + t * Rc, Rc), pl.ds(c0, CB)], sem)

    def wait_out(t, buf, sem):
        pltpu.make_async_copy(
            buf, out_hbm.at[pl.ds(rbase + t * Rc, Rc), pl.ds(c0, CB)], sem).wait()

    def compute(xbuf, obuf):
        @plsc.parallel_loop(0, VPC, step=1, unroll=4)
        def vec_body(i):
            row = lax.shift_right_logical(i, 3)
            cofs = pl.multiple_of((i & 7) * L, L)
            xv = xbuf[row, pl.ds(cofs, L)]
            K = lanes + cofs
            for b in (64, 32, 16, 8, 4, 2, 1):
                val = plsc.load_gather(yr_tab, [K + ((b - 1) * CB)])
                K = jnp.where(val < xv, K + b * CB, K)
            ykx = plsc.load_gather(yk_tab, [K])
            A = plsc.load_gather(a_tab, [K])
            B = plsc.load_gather(b_tab, [K])
            H = plsc.load_gather(h_tab, [K])
            XS = plsc.load_gather(xs_tab, [K])
            xr = xv - ykx
            dm = jnp.maximum(B * xr + A * A, 1e-30)
            iv = 0x1F3D1DF5 + lax.shift_right_logical(
                lax.bitcast_convert_type(dm, jnp.int32), 1)
            s0 = lax.bitcast_convert_type(iv, jnp.float32)
            s1 = s0 + (0.25 * dm) / s0
            obuf[row, pl.ds(cofs, L)] = (H * xr) / (s1 + A) + XS

    xb = (xbufA, xbufB)
    ob = (obufA, obufB)
    isem = (isemA, isemB)
    osem = (osemA, osemB)
    start_in(0, xb[0], isem[0])

    def chunk_pair(g, carry):
        t0 = g * 2
        for ph in (0, 1):
            t = t0 + ph
            nxt = t + 1

            @pl.when(nxt < NCHUNK)
            def _():
                start_in(nxt, xb[1 - ph], isem[1 - ph])

            wait_in(t, xb[ph], isem[ph])

            @pl.when(t >= 2)
            def _():
                wait_out(t, ob[ph], osem[ph])

            compute(xb[ph], ob[ph])
            start_out(t, ob[ph], osem[ph])
        return carry

    lax.fori_loop(0, NCHUNK // 2, chunk_pair, 0)
    wait_out(NCHUNK - 2, ob[0], osem[0])
    wait_out(NCHUNK - 1, ob[1], osem[1])


_sc_call = pl.kernel(
    _body,
    out_type=jax.ShapeDtypeStruct((N, INPUT_DIM), jnp.float32),
    mesh=plsc.VectorSubcoreMesh(core_axis_name="c", subcore_axis_name="s"),
    compiler_params=pltpu.CompilerParams(needs_layout_passes=False),
    scratch_types=[
        pltpu.VMEM((YR_ROWS * CB,), jnp.float32),
        pltpu.VMEM((KROWS * CB,), jnp.float32),
        pltpu.VMEM((KROWS * CB,), jnp.float32),
        pltpu.VMEM((KROWS * CB,), jnp.float32),
        pltpu.VMEM((KROWS * CB,), jnp.float32),
        pltpu.VMEM((KROWS * CB,), jnp.float32),
        pltpu.VMEM((Rc, CB), jnp.float32),
        pltpu.VMEM((Rc, CB), jnp.float32),
        pltpu.VMEM((Rc, CB), jnp.float32),
        pltpu.VMEM((Rc, CB), jnp.float32),
        pltpu.SemaphoreType.DMA,
        pltpu.SemaphoreType.DMA,
        pltpu.SemaphoreType.DMA,
        pltpu.SemaphoreType.DMA,
    ],
)


def kernel(x, log_p):
    xs_c = jnp.asarray(np.tile(_XS66[:, None], (1, CB)).reshape(-1))
    h_c = jnp.asarray(np.tile(_H66[:, None], (1, CB)).reshape(-1))
    return _sc_call(x, log_p, xs_c, h_c)
